# Initial kernel scaffold; baseline (speedup 1.0000x reference)
import jax
import jax.numpy as jnp
from jax.experimental import pallas as pl


def kernel(input_ids, table, W, b):
    raise NotImplementedError("write your pallas kernel here")


# speedup vs baseline: 4.5413x; 4.5413x over previous
import functools

import jax
import jax.numpy as jnp
from jax import lax
from jax.experimental import pallas as pl
from jax.experimental.pallas import tpu as pltpu
from jax.experimental.pallas import tpu_sc as plsc


def _transform_body(table_ref, w_ref, b_ref, out_ref):
    out_ref[...] = (
        lax.dot_general(
            table_ref[...],
            w_ref[...],
            (((1,), (1,)), ((), ())),
            preferred_element_type=jnp.float32,
        )
        + b_ref[...]
    )


def _make_gather(n_rows: int, v_pad: int, d: int, d_pad: int, chunk: int):
    mesh = plsc.VectorSubcoreMesh(core_axis_name="c", subcore_axis_name="s")
    nw = mesh.num_cores * mesh.num_subcores
    assert n_rows % (nw * chunk) == 0
    rows_per_w = n_rows // nw
    n_chunks = rows_per_w // chunk

    @functools.partial(
        pl.kernel,
        mesh=mesh,
        out_type=jax.ShapeDtypeStruct((n_rows, d), jnp.float32),
        scratch_types=[
            pltpu.VMEM((v_pad, d_pad), jnp.float32),
            pltpu.VMEM((chunk,), jnp.int32),
            pltpu.VMEM((chunk, d), jnp.float32),
        ],
    )
    def gather(t2_hbm, idx_hbm, out_hbm, t2_v, idx_v, rows_v):
        wid = lax.axis_index("s") * mesh.num_cores + lax.axis_index("c")
        base = wid * rows_per_w
        pltpu.sync_copy(t2_hbm, t2_v)

        @pl.loop(0, n_chunks)
        def _chunk(i):
            off = base + i * chunk
            pltpu.sync_copy(idx_hbm.at[pl.ds(off, chunk)], idx_v)

            @pl.loop(0, chunk // 16)
            def _grp(g):
                ids16 = idx_v[pl.ds(g * 16, 16)]
                for j in range(16):
                    rid = ids16[j]
                    r = g * 16 + j
                    for c in range(0, d, 16):
                        rows_v[r, pl.ds(c, 16)] = t2_v[rid, pl.ds(c, 16)]

            pltpu.sync_copy(rows_v, out_hbm.at[pl.ds(off, chunk)])

    return gather


def kernel(input_ids, table, W, b):
    bsz, seq = input_ids.shape
    v, d = table.shape
    n_rows = bsz * seq

    v_pad = (v + 7) // 8 * 8
    d_pad = 128
    table_pad = jnp.zeros((v_pad, d), jnp.float32).at[:v].set(table)
    w_pad = jnp.zeros((d_pad, d), jnp.float32).at[:d].set(W)
    b_pad = jnp.zeros((1, d_pad), jnp.float32).at[0, :d].set(b)

    t2 = pl.pallas_call(
        _transform_body,
        out_shape=jax.ShapeDtypeStruct((v_pad, d_pad), jnp.float32),
    )(table_pad, w_pad, b_pad)

    ids_flat = input_ids.reshape(n_rows).astype(jnp.int32)
    out_flat = _make_gather(n_rows, v_pad, d, d_pad, chunk=512)(t2, ids_flat)
    return out_flat.reshape(bsz, seq, d)


# speedup vs baseline: 5.8415x; 1.2863x over previous
import functools

import jax
import jax.numpy as jnp
from jax import lax
from jax.experimental import pallas as pl
from jax.experimental.pallas import tpu as pltpu
from jax.experimental.pallas import tpu_sc as plsc


def _transform_body(table_ref, w_ref, b_ref, out_ref):
    out_ref[...] = (
        lax.dot_general(
            table_ref[...],
            w_ref[...],
            (((1,), (1,)), ((), ())),
            preferred_element_type=jnp.float32,
        )
        + b_ref[...]
    )


def _make_gather(n_rows: int, v_pad: int, d: int, d_pad: int, chunk: int):
    mesh = plsc.VectorSubcoreMesh(core_axis_name="c", subcore_axis_name="s")
    nw = mesh.num_cores * mesh.num_subcores
    assert n_rows % (nw * chunk) == 0
    rows_per_w = n_rows // nw
    n_chunks = rows_per_w // chunk

    assert n_chunks % 2 == 0

    @functools.partial(
        pl.kernel,
        mesh=mesh,
        out_type=jax.ShapeDtypeStruct((n_rows, d), jnp.float32),
        scratch_types=[
            pltpu.VMEM((v_pad, d_pad), jnp.float32),
            pltpu.VMEM((2 * chunk,), jnp.int32),
            pltpu.VMEM((chunk, d), jnp.float32),
            pltpu.VMEM((chunk, d), jnp.float32),
            pltpu.SemaphoreType.DMA,
            pltpu.SemaphoreType.DMA,
            pltpu.SemaphoreType.DMA,
        ],
    )
    def gather(t2_hbm, idx_hbm, out_hbm, t2_v, idx_v, rows0, rows1, isem,
               osem0, osem1):
        wid = lax.axis_index("s") * mesh.num_cores + lax.axis_index("c")
        base = wid * rows_per_w
        rowss = (rows0, rows1)
        osems = (osem0, osem1)
        pltpu.sync_copy(t2_hbm, t2_v)

        def idx_start(g, slot):
            pltpu.async_copy(
                idx_hbm.at[pl.ds(base + g * chunk, chunk)],
                idx_v.at[pl.ds(slot * chunk, chunk)], isem)

        def idx_wait(slot):
            pltpu.make_async_copy(
                idx_hbm.at[pl.ds(base, chunk)], idx_v.at[pl.ds(slot * chunk, chunk)], isem).wait()

        idx_start(0, 0)

        @pl.loop(0, n_chunks, step=2)
        def _pair(g):
            for s in range(2):
                gg = g + s
                out_slice = out_hbm.at[pl.ds(base + gg * chunk, chunk)]

                @pl.when(gg + 1 < n_chunks)
                def _prefetch():
                    idx_start(gg + 1, s ^ 1)

                idx_wait(s)

                @pl.when(gg >= 2)
                def _drain():
                    pltpu.make_async_copy(rowss[s], out_slice, osems[s]).wait()

                @pl.loop(0, chunk // 16)
                def _grp(gr):
                    ids16 = idx_v[pl.ds(s * chunk + gr * 16, 16)]
                    for j in range(16):
                        rid = ids16[j]
                        r = gr * 16 + j
                        for c in range(0, d, 16):
                            rowss[s][r, pl.ds(c, 16)] = t2_v[rid, pl.ds(c, 16)]

                pltpu.async_copy(rowss[s], out_slice, osems[s])

        for s in range(2):
            pltpu.make_async_copy(
                rowss[s], out_hbm.at[pl.ds(base, chunk)], osems[s]).wait()

    return gather


def kernel(input_ids, table, W, b):
    bsz, seq = input_ids.shape
    v, d = table.shape
    n_rows = bsz * seq

    v_pad = (v + 7) // 8 * 8
    d_pad = 128
    table_pad = jnp.zeros((v_pad, d), jnp.float32).at[:v].set(table)
    w_pad = jnp.zeros((d_pad, d), jnp.float32).at[:d].set(W)
    b_pad = jnp.zeros((1, d_pad), jnp.float32).at[0, :d].set(b)

    t2 = pl.pallas_call(
        _transform_body,
        out_shape=jax.ShapeDtypeStruct((v_pad, d_pad), jnp.float32),
    )(table_pad, w_pad, b_pad)

    ids_flat = input_ids.reshape(n_rows).astype(jnp.int32)
    out_flat = _make_gather(n_rows, v_pad, d, d_pad, chunk=400)(t2, ids_flat)
    return out_flat.reshape(bsz, seq, d)


# speedup vs baseline: 7.3973x; 1.2663x over previous
import functools

import jax
import jax.numpy as jnp
from jax import lax
from jax.experimental import pallas as pl
from jax.experimental.pallas import tpu as pltpu
from jax.experimental.pallas import tpu_sc as plsc


def _transform_body(w_ref, table_ref, b_ref, out_ref):
    out_ref[...] = (
        lax.dot_general(
            w_ref[...],
            table_ref[...],
            (((1,), (1,)), ((), ())),
            preferred_element_type=jnp.float32,
        )
        + b_ref[...]
    )


def _make_gather(seq: int, n_b: int, d: int, v_pad: int):
    mesh = plsc.VectorSubcoreMesh(core_axis_name="c", subcore_axis_name="s")
    nw = mesh.num_cores * mesh.num_subcores
    assert n_b % nw == 0 and seq % 2 == 0
    b_per_w = n_b // nw

    @functools.partial(
        pl.kernel,
        mesh=mesh,
        out_type=jax.ShapeDtypeStruct((seq, d, n_b), jnp.float32),
        scratch_types=[
            pltpu.VMEM((d * v_pad,), jnp.float32),
            pltpu.VMEM((2 * b_per_w,), jnp.int32),
            pltpu.VMEM((d, b_per_w), jnp.float32),
            pltpu.VMEM((d, b_per_w), jnp.float32),
            pltpu.SemaphoreType.DMA,
            pltpu.SemaphoreType.DMA,
            pltpu.SemaphoreType.DMA,
        ],
        compiler_params=pltpu.CompilerParams(needs_layout_passes=False),
    )
    def gather(t2t_hbm, idx_hbm, out_hbm, t2t_v, idx_v, slab0, slab1, isem,
               osem0, osem1):
        wid = lax.axis_index("s") * mesh.num_cores + lax.axis_index("c")
        b0 = wid * b_per_w
        slabs = (slab0, slab1)
        osems = (osem0, osem1)
        pltpu.sync_copy(t2t_hbm, t2t_v)

        def idx_start(l, slot):
            pltpu.async_copy(
                idx_hbm.at[l, pl.ds(b0, b_per_w)],
                idx_v.at[pl.ds(slot * b_per_w, b_per_w)], isem)

        def idx_wait(slot):
            pltpu.make_async_copy(
                idx_hbm.at[0, pl.ds(b0, b_per_w)],
                idx_v.at[pl.ds(slot * b_per_w, b_per_w)], isem).wait()

        idx_start(0, 0)

        @pl.loop(0, seq, step=2)
        def _pair(l):
            for s in range(2):
                ll = l + s
                out_slice = out_hbm.at[ll, :, pl.ds(b0, b_per_w)]

                @pl.when(ll + 1 < seq)
                def _prefetch():
                    idx_start(ll + 1, s ^ 1)

                idx_wait(s)

                @pl.when(ll >= 2)
                def _drain():
                    pltpu.make_async_copy(slabs[s], out_slice, osems[s]).wait()

                @pl.loop(0, b_per_w // 16)
                def _grp(g):
                    ids16 = idx_v[pl.ds(s * b_per_w + g * 16, 16)]
                    for o in range(d):
                        vals = plsc.load_gather(t2t_v, [ids16 + o * v_pad])
                        slabs[s][o, pl.ds(g * 16, 16)] = vals

                pltpu.async_copy(slabs[s], out_slice, osems[s])

        for s in range(2):
            pltpu.make_async_copy(
                slabs[s], out_hbm.at[0, :, pl.ds(b0, b_per_w)], osems[s]).wait()

    return gather


def kernel(input_ids, table, W, b):
    bsz, seq = input_ids.shape
    v, d = table.shape

    v_pad = 128
    table_pad = jnp.zeros((v_pad, d), jnp.float32).at[:v].set(table)

    t2t = pl.pallas_call(
        _transform_body,
        out_shape=jax.ShapeDtypeStruct((d, v_pad), jnp.float32),
    )(W, table_pad, jnp.reshape(b, (d, 1)))

    ids_t = input_ids.T.astype(jnp.int32)
    out_lob = _make_gather(seq, bsz, d, v_pad)(t2t.reshape(d * v_pad), ids_t)
    return jnp.transpose(out_lob, (2, 0, 1))
